# baseline (speedup 1.0000x reference)
import jax
import jax.numpy as jnp
from jax import lax
from jax.experimental import pallas as pl
from jax.experimental.pallas import tpu as pltpu
from jax.experimental.pallas import tpu_sc as plsc

VOCAB = 100000
EMBED = 64
BATCH = 1024
CTX = 20

NC = 2
NS = 16
NW = NC * NS
ROWS_PER_W = BATCH * CTX // NW
CHUNK = 128
NCHUNK = ROWS_PER_W // CHUNK


def _sc_gather_body(x_hbm, table_hbm, e_hbm, idx_v, rows_v, sem):
    wid = lax.axis_index("s") * NC + lax.axis_index("c")
    base = wid * ROWS_PER_W
    pltpu.sync_copy(x_hbm.at[wid], idx_v)
    copies = [
        pltpu.async_copy(
            table_hbm.at[idx_v.at[j]],
            rows_v.at[pl.ds(j * CHUNK, CHUNK)],
            sem,
        )
        for j in range(NCHUNK)
    ]
    for c in copies:
        c.wait()
    pltpu.sync_copy(rows_v, e_hbm.at[pl.ds(base, ROWS_PER_W)])


import functools


@functools.cache
def _make_sc_gather():
    return pl.kernel(
        _sc_gather_body,
        out_type=jax.ShapeDtypeStruct((BATCH * CTX, EMBED), jnp.float32),
        mesh=plsc.VectorSubcoreMesh(core_axis_name="c", subcore_axis_name="s"),
        compiler_params=pltpu.CompilerParams(use_tc_tiling_on_sc=False),
        scratch_types=[
            pltpu.VMEM((NCHUNK, CHUNK), jnp.int32),
            pltpu.VMEM((ROWS_PER_W, EMBED), jnp.float32),
            pltpu.SemaphoreType.DMA,
        ],
    )


def _pool_body(e_ref, h_ref):
    acc = jnp.zeros((BATCH, EMBED), jnp.float32)
    for j in range(CTX):
        row = e_ref[:, j, :]
        sumsq = jnp.sum(row * row, axis=-1, keepdims=True)
        norm = jnp.sqrt(sumsq)
        scale = jnp.where(norm > 1.0, 1.0 / (norm + 1e-7), 1.0)
        acc = acc + row * scale
    h_ref[...] = acc * (1.0 / CTX)


def _pool(e3):
    return pl.pallas_call(
        _pool_body,
        out_shape=jax.ShapeDtypeStruct((BATCH, EMBED), jnp.float32),
    )(e3)


VT = 2048
GRID_V = 49
VOCAB_P = GRID_V * VT
VT_LAST = VOCAB - (GRID_V - 1) * VT
NBUF = 5


def _full_copy(acc, out_hbm, sems, j):
    return pltpu.make_async_copy(
        acc.at[j % NBUF],
        out_hbm.at[:, pl.ds(j * VT, VT)],
        sems.at[j % NBUF],
    )


def _matmul_body(h_ref, wt_ref, b_ref, out_hbm, acc, acc_last, sems, sem_last):
    i = pl.program_id(0)
    slot = lax.rem(i, NBUF)

    @pl.when(i >= NBUF)
    def _wait_prev():
        pltpu.make_async_copy(
            acc.at[slot],
            out_hbm.at[:, pl.ds((i - NBUF) * VT, VT)],
            sems.at[slot],
        ).wait()

    out = lax.dot_general(
        h_ref[...], wt_ref[0],
        (((1,), (0,)), ((), ())),
        preferred_element_type=jnp.float32,
    ) + b_ref[0]

    @pl.when(i < GRID_V - 1)
    def _start_full():
        acc[slot] = out
        pltpu.make_async_copy(
            acc.at[slot],
            out_hbm.at[:, pl.ds(i * VT, VT)],
            sems.at[slot],
        ).start()

    @pl.when(i == GRID_V - 1)
    def _last_and_drain():
        acc_last[...] = out[:, :VT_LAST]
        pltpu.make_async_copy(
            acc_last,
            out_hbm.at[:, pl.ds((GRID_V - 1) * VT, VT_LAST)],
            sem_last,
        ).start()
        for k in range(NBUF - 1):
            _full_copy(acc, out_hbm, sems, GRID_V - NBUF + k).wait()
        pltpu.make_async_copy(
            acc_last,
            out_hbm.at[:, pl.ds((GRID_V - 1) * VT, VT_LAST)],
            sem_last,
        ).wait()


def _matmul(h, wt, b2):
    return pl.pallas_call(
        _matmul_body,
        out_shape=jax.ShapeDtypeStruct((BATCH, VOCAB), jnp.float32),
        grid=(GRID_V,),
        in_specs=[
            pl.BlockSpec((BATCH, EMBED), lambda i: (0, 0)),
            pl.BlockSpec((1, EMBED, VT), lambda i: (i, 0, 0)),
            pl.BlockSpec((1, 1, VT), lambda i: (i, 0, 0)),
        ],
        out_specs=pl.BlockSpec(memory_space=pl.ANY),
        scratch_shapes=[
            pltpu.VMEM((NBUF, BATCH, VT), jnp.float32),
            pltpu.VMEM((BATCH, VT_LAST), jnp.float32),
            pltpu.SemaphoreType.DMA((NBUF,)),
            pltpu.SemaphoreType.DMA,
        ],
    )(h, wt, b2)


def kernel(x, emb_table, W, b):
    x_flat = x.astype(jnp.int32).reshape(NW, NCHUNK, CHUNK)
    e = _make_sc_gather()(x_flat, emb_table)
    h = _pool(e.reshape(BATCH, CTX, EMBED))
    pad = VOCAB_P - VOCAB
    wt3 = jnp.pad(W, ((0, pad), (0, 0))).reshape(GRID_V, VT, EMBED).transpose(0, 2, 1)
    b3 = jnp.pad(b, (0, pad)).reshape(GRID_V, 1, VT)
    return _matmul(h, wt3, b3)


# speedup vs baseline: 1.0399x; 1.0399x over previous
import functools

import jax
import jax.numpy as jnp
from jax import lax
from jax.experimental import pallas as pl
from jax.experimental.pallas import tpu as pltpu
from jax.experimental.pallas import tpu_sc as plsc

VOCAB = 100000
EMBED = 64
BATCH = 1024
CTX = 20

NC = 2
NS = 16
NW = NC * NS
B_PER_W = BATCH // NW
NCHUNK = EMBED // 16


def _rsqrt16(s):
    i = plsc.bitcast(s, jnp.int32)
    y = plsc.bitcast(
        jnp.full((16,), 0x5F3759DF, jnp.int32) - lax.shift_right_logical(i, 1),
        jnp.float32,
    )
    for _ in range(3):
        y = y * (1.5 - 0.5 * s * y * y)
    return y


def _sc_pool_body(x_hbm, table_hbm, h_hbm, idx_v, rows_v, h_v, sem):
    wid = lax.axis_index("s") * NC + lax.axis_index("c")
    base_b = wid * B_PER_W
    pltpu.sync_copy(x_hbm.at[pl.ds(base_b, B_PER_W)], idx_v)
    copies = [
        pltpu.async_copy(table_hbm.at[idx_v.at[b]], rows_v.at[b], sem)
        for b in range(B_PER_W)
    ]
    for c in copies:
        c.wait()
    for b in range(B_PER_W):
        def ctx_step(j, carry):
            chunks = [rows_v[b, j, pl.ds(16 * k, 16)] for k in range(NCHUNK)]
            q = chunks[0] * chunks[0]
            for k in range(1, NCHUNK):
                q = q + chunks[k] * chunks[k]
            s = jnp.full((16,), jnp.sum(q), jnp.float32)
            scale = jnp.where(s > 1.0, _rsqrt16(s), 1.0)
            return tuple(a + c * scale for a, c in zip(carry, chunks))

        acc = lax.fori_loop(
            0, CTX, ctx_step,
            tuple(jnp.zeros((16,), jnp.float32) for _ in range(NCHUNK)),
        )
        for k in range(NCHUNK):
            h_v[b, pl.ds(16 * k, 16)] = acc[k] * (1.0 / CTX)
    pltpu.sync_copy(h_v, h_hbm.at[pl.ds(base_b, B_PER_W)])


@functools.cache
def _make_sc_pool():
    return pl.kernel(
        _sc_pool_body,
        out_type=jax.ShapeDtypeStruct((BATCH, EMBED), jnp.float32),
        mesh=plsc.VectorSubcoreMesh(core_axis_name="c", subcore_axis_name="s"),
        compiler_params=pltpu.CompilerParams(
            use_tc_tiling_on_sc=False, needs_layout_passes=False,
        ),
        scratch_types=[
            pltpu.VMEM((B_PER_W, CTX), jnp.int32),
            pltpu.VMEM((B_PER_W, CTX, EMBED), jnp.float32),
            pltpu.VMEM((B_PER_W, EMBED), jnp.float32),
            pltpu.SemaphoreType.DMA,
        ],
    )


VT = 2048
GRID_V = 49
VOCAB_P = GRID_V * VT
VT_LAST = VOCAB - (GRID_V - 1) * VT
NBUF = 5


def _full_copy(acc, out_hbm, sems, j):
    return pltpu.make_async_copy(
        acc.at[j % NBUF],
        out_hbm.at[:, pl.ds(j * VT, VT)],
        sems.at[j % NBUF],
    )


def _matmul_body(h_ref, wt_ref, b_ref, out_hbm, acc, acc_last, sems, sem_last):
    i = pl.program_id(0)
    slot = lax.rem(i, NBUF)

    @pl.when(i >= NBUF)
    def _wait_prev():
        pltpu.make_async_copy(
            acc.at[slot],
            out_hbm.at[:, pl.ds((i - NBUF) * VT, VT)],
            sems.at[slot],
        ).wait()

    out = lax.dot_general(
        h_ref[...], wt_ref[0],
        (((1,), (0,)), ((), ())),
        preferred_element_type=jnp.float32,
    ) + b_ref[0]

    @pl.when(i < GRID_V - 1)
    def _start_full():
        acc[slot] = out
        pltpu.make_async_copy(
            acc.at[slot],
            out_hbm.at[:, pl.ds(i * VT, VT)],
            sems.at[slot],
        ).start()

    @pl.when(i == GRID_V - 1)
    def _last_and_drain():
        acc_last[...] = out[:, :VT_LAST]
        pltpu.make_async_copy(
            acc_last,
            out_hbm.at[:, pl.ds((GRID_V - 1) * VT, VT_LAST)],
            sem_last,
        ).start()
        for k in range(NBUF - 1):
            _full_copy(acc, out_hbm, sems, GRID_V - NBUF + k).wait()
        pltpu.make_async_copy(
            acc_last,
            out_hbm.at[:, pl.ds((GRID_V - 1) * VT, VT_LAST)],
            sem_last,
        ).wait()


def _matmul(h, wt, b2):
    return pl.pallas_call(
        _matmul_body,
        out_shape=jax.ShapeDtypeStruct((BATCH, VOCAB), jnp.float32),
        grid=(GRID_V,),
        in_specs=[
            pl.BlockSpec((BATCH, EMBED), lambda i: (0, 0)),
            pl.BlockSpec((1, EMBED, VT), lambda i: (i, 0, 0)),
            pl.BlockSpec((1, 1, VT), lambda i: (i, 0, 0)),
        ],
        out_specs=pl.BlockSpec(memory_space=pl.ANY),
        scratch_shapes=[
            pltpu.VMEM((NBUF, BATCH, VT), jnp.float32),
            pltpu.VMEM((BATCH, VT_LAST), jnp.float32),
            pltpu.SemaphoreType.DMA((NBUF,)),
            pltpu.SemaphoreType.DMA,
        ],
    )(h, wt, b2)


def kernel(x, emb_table, W, b):
    h = _make_sc_pool()(x.astype(jnp.int32), emb_table)
    pad = VOCAB_P - VOCAB
    wt3 = jnp.pad(W, ((0, pad), (0, 0))).reshape(GRID_V, VT, EMBED).transpose(0, 2, 1)
    b3 = jnp.pad(b, (0, pad)).reshape(GRID_V, 1, VT)
    return _matmul(h, wt3, b3)
